# baseline (speedup 1.0000x reference)
import functools

import jax
import jax.numpy as jnp
from jax import lax
from jax.experimental import pallas as pl
from jax.experimental.pallas import tpu as pltpu
from jax.experimental.pallas import tpu_sc as plsc

_NUM_CORES = 2
_NUM_SUBCORES = 16
_NUM_WORKERS = _NUM_CORES * _NUM_SUBCORES
_LANES = 16
_SHIFT = 2


@functools.partial(jax.jit, static_argnums=(2, 3, 4))
def _position_embedding(row_embed, col_embed, b, h, w):
    d = row_embed.shape[-1]
    nch = 2 * d
    assert h == _NUM_WORKERS and d % _LANES == 0

    mesh = plsc.VectorSubcoreMesh(core_axis_name="c", subcore_axis_name="s")

    @functools.partial(
        pl.kernel,
        mesh=mesh,
        out_type=jax.ShapeDtypeStruct((b, h, w, nch), jnp.float32),
        scratch_types=[
            pltpu.VMEM((w, nch), jnp.float32),
            pltpu.VMEM((w, nch), jnp.float32),
            pltpu.VMEM((1, d), jnp.float32),
            pltpu.VMEM((1, d), jnp.float32),
            pltpu.SemaphoreType.DMA,
            pltpu.SemaphoreType.DMA,
        ],
    )
    def sc_kernel(row_hbm, col_hbm, out_hbm,
                  slab_v, slab2_v, row_v, row2_v, sem, sem2):
        cid = lax.axis_index("c")
        sid = lax.axis_index("s")
        y = cid * _NUM_SUBCORES + sid
        col_cp = pltpu.make_async_copy(
            col_hbm.at[pl.ds(0, w)], slab_v.at[:, pl.ds(0, d)], sem)
        row_cp = pltpu.make_async_copy(row_hbm.at[pl.ds(y, 1)], row_v, sem)
        col_cp.start()
        row_cp.start()

        @pl.when(cid == 1)
        def _stage_partner():
            pltpu.make_async_copy(
                col_hbm.at[pl.ds(0, w)], slab2_v.at[:, pl.ds(0, d)],
                sem2).start()
            pltpu.make_async_copy(
                row_hbm.at[pl.ds(sid, 1)], row2_v, sem2).start()

        col_cp.wait()
        row_cp.wait()

        def _mk_fill(slab, rowbuf):
            gs = [rowbuf[0, pl.ds(j * _LANES, _LANES)]
                  for j in range(d // _LANES)]

            def _fill(xi, carry):
                for j, g in enumerate(gs):
                    slab[xi, pl.ds(d + j * _LANES, _LANES)] = g
                return carry

            return _fill

        lax.fori_loop(0, w, _mk_fill(slab_v, row_v), 0)

        @pl.when(cid == 1)
        def _build_partner():
            pltpu.make_async_copy(
                col_hbm.at[pl.ds(0, w)], slab2_v.at[:, pl.ds(0, d)],
                sem2).wait()
            pltpu.make_async_copy(
                row_hbm.at[pl.ds(sid, 1)], row2_v, sem2).wait()
            lax.fori_loop(0, w, _mk_fill(slab2_v, row2_v), 0)

        nb = b - _SHIFT * (1 - cid)

        def _fire(bi, carry):
            pltpu.make_async_copy(slab_v, out_hbm.at[bi, y], sem).start()
            return carry

        def _drain(bi, carry):
            pltpu.make_async_copy(slab_v, out_hbm.at[bi, y], sem).wait()
            return carry

        def _fire2(bi, carry):
            pltpu.make_async_copy(slab2_v, out_hbm.at[bi, sid], sem).start()
            return carry

        def _drain2(bi, carry):
            pltpu.make_async_copy(slab2_v, out_hbm.at[bi, sid], sem).wait()
            return carry

        lax.fori_loop(0, nb, _fire, 0)

        @pl.when(cid == 1)
        def _fire_partner():
            lax.fori_loop(b - _SHIFT, b, _fire2, 0)

        lax.fori_loop(0, nb, _drain, 0)

        @pl.when(cid == 1)
        def _drain_partner():
            lax.fori_loop(b - _SHIFT, b, _drain2, 0)

    out = sc_kernel(row_embed, col_embed)
    return jnp.transpose(out, (0, 3, 1, 2))


def kernel(x, row_embed, col_embed):
    b = x.shape[0]
    h, w = x.shape[-2], x.shape[-1]
    return _position_embedding(row_embed, col_embed, b, h, w)


# speedup vs baseline: 1.0666x; 1.0666x over previous
import functools

import jax
import jax.numpy as jnp
from jax import lax
from jax.experimental import pallas as pl
from jax.experimental.pallas import tpu as pltpu
from jax.experimental.pallas import tpu_sc as plsc

_NUM_CORES = 2
_NUM_SUBCORES = 16
_NUM_WORKERS = _NUM_CORES * _NUM_SUBCORES
_LANES = 16


@functools.partial(jax.jit, static_argnums=(2, 3, 4))
def _position_embedding(row_embed, col_embed, b, h, w):
    d = row_embed.shape[-1]
    nch = 2 * d
    assert h == _NUM_WORKERS and d % _LANES == 0

    mesh = plsc.VectorSubcoreMesh(core_axis_name="c", subcore_axis_name="s")

    @functools.partial(
        pl.kernel,
        mesh=mesh,
        out_type=jax.ShapeDtypeStruct((b, h, w, nch), jnp.float32),
        scratch_types=[
            pltpu.VMEM((w, nch), jnp.float32),
            pltpu.VMEM((1, d), jnp.float32),
            pltpu.SemaphoreType.DMA,
        ],
    )
    def sc_kernel(row_hbm, col_hbm, out_hbm, slab_v, row_v, sem):
        y = lax.axis_index("s") * _NUM_CORES + lax.axis_index("c")
        col_cp = pltpu.make_async_copy(
            col_hbm.at[pl.ds(0, w)], slab_v.at[:, pl.ds(0, d)], sem)
        row_cp = pltpu.make_async_copy(row_hbm.at[pl.ds(y, 1)], row_v, sem)
        col_cp.start()
        row_cp.start()
        col_cp.wait()
        row_cp.wait()

        gs = [row_v[0, pl.ds(j * _LANES, _LANES)] for j in range(d // _LANES)]

        def _fill(xi, carry):
            for j, g in enumerate(gs):
                slab_v[xi, pl.ds(d + j * _LANES, _LANES)] = g
            return carry

        lax.fori_loop(0, w, _fill, 0)

        def _fire(bi, carry):
            pltpu.make_async_copy(slab_v, out_hbm.at[bi, y], sem).start()
            return carry

        def _drain(bi, carry):
            pltpu.make_async_copy(slab_v, out_hbm.at[bi, y], sem).wait()
            return carry

        lax.fori_loop(0, b, _fire, 0)
        lax.fori_loop(0, b, _drain, 0)

    out = sc_kernel(row_embed, col_embed)
    return jnp.transpose(out, (0, 3, 1, 2))


def kernel(x, row_embed, col_embed):
    b = x.shape[0]
    h, w = x.shape[-2], x.shape[-1]
    return _position_embedding(row_embed, col_embed, b, h, w)
